# baseline (speedup 1.0000x reference)
import functools

import jax
import jax.numpy as jnp
from jax import lax
from jax.experimental import pallas as pl
from jax.experimental.pallas import tpu as pltpu
from jax.experimental.pallas import tpu_sc as plsc

N = 10000
E = 320000
D = 128
G = 16

NCORE = 2
NSUB = 16
NT = NCORE * NSUB

K = 128
NCH = 80
TCH = 2560
GRP = 8
NCH0 = 160
SUPS0 = (40, 40, 40, 40)
SGMAX = 40
EPT = K * NCH
E_PAD = EPT * NT
N_PAD = 10112
RZ = N_PAD // NSUB

BLK = 1000
GRID = N // BLK


def _mesh():
    return plsc.VectorSubcoreMesh(core_axis_name="c", subcore_axis_name="s")


def _sc_deg(dst3, z128):
    @functools.partial(
        pl.kernel,
        mesh=_mesh(),
        out_type=jax.ShapeDtypeStruct((NCORE, N_PAD, D), jnp.float32),
        scratch_types=[
            pltpu.VMEM((NCH, K), jnp.int32),
            pltpu.VMEM((K, D), jnp.float32),
            pltpu.VMEM_SHARED((N_PAD, D), jnp.float32),
        ],
    )
    def k(dst_hbm, z_hbm, out_hbm, dst_v, ones_v, acc):
        cid = lax.axis_index("c")
        sid = lax.axis_index("s")
        wid = sid * NCORE + cid
        pltpu.sync_copy(z_hbm, acc.at[pl.ds(sid * RZ, RZ)])

        def fill(r, carry):
            for g in range(D // 16):
                ones_v[r, pl.ds(g * 16, 16)] = jnp.full((16,), 1.0,
                                                        jnp.float32)
            return carry

        lax.fori_loop(0, K, fill, 0)
        pltpu.sync_copy(dst_hbm.at[pl.ds(wid * NCH, NCH)], dst_v)
        plsc.subcore_barrier()

        def body(c, carry):
            pltpu.sync_copy(ones_v, acc.at[dst_v.at[c]], add=True)
            return carry

        lax.fori_loop(0, NCH, body, 0)
        plsc.subcore_barrier()
        pltpu.sync_copy(acc.at[pl.ds(sid * RZ, RZ)],
                        out_hbm.at[cid, pl.ds(sid * RZ, RZ)])

    return k(dst3, z128)


def _sc_agg(hs_pad, src3, dst3, z128):
    @functools.partial(
        pl.kernel,
        mesh=_mesh(),
        out_type=jax.ShapeDtypeStruct((1, N_PAD, D), jnp.float32),
        scratch_types=[
            pltpu.VMEM((SGMAX, K), jnp.int32),
            pltpu.VMEM((SGMAX, K), jnp.int32),
            pltpu.VMEM((K, D), jnp.float32),
            pltpu.VMEM((K, D), jnp.float32),
            pltpu.VMEM_SHARED((N_PAD, D), jnp.float32),
            pltpu.SemaphoreType.DMA,
            pltpu.SemaphoreType.DMA,
        ],
    )
    def k(hs_hbm, src_hbm, dst_hbm, z_hbm, out_hbm,
          ssup, dsup, r0, r1, acc, s0, s1):
        cid = lax.axis_index("c")
        sid = lax.axis_index("s")

        @pl.when(cid == 0)
        def _():
            pltpu.sync_copy(z_hbm, acc.at[pl.ds(sid * RZ, RZ)])

        plsc.subcore_barrier()

        rows = (r0, r1)
        sems = (s0, s1)

        def run_part(base, supers):
            off = 0
            for sg in supers:
                sb = base + off
                pltpu.sync_copy(src_hbm.at[pl.ds(sb, sg)],
                                ssup.at[pl.ds(0, sg)])
                pltpu.sync_copy(dst_hbm.at[pl.ds(sb, sg)],
                                dsup.at[pl.ds(0, sg)])
                pltpu.async_copy(hs_hbm.at[ssup.at[0]], r0, s0)
                pltpu.async_copy(hs_hbm.at[ssup.at[1]], r1, s1)

                def pair(j, carry):
                    for b in range(2):
                        c = 2 * j + b
                        pltpu.make_async_copy(hs_hbm.at[ssup.at[c]],
                                              rows[b], sems[b]).wait()
                        pltpu.sync_copy(rows[b], acc.at[dsup.at[c]],
                                        add=True)

                        @pl.when(c + 2 < sg)
                        def _():
                            pltpu.async_copy(hs_hbm.at[ssup.at[c + 2]],
                                             rows[b], sems[b])
                    return carry

                lax.fori_loop(0, sg // 2, pair, 0)
                off += sg

        @pl.when(cid == 0)
        def _():
            run_part(sid * NCH0, SUPS0)

        plsc.subcore_barrier()

        @pl.when(cid == 0)
        def _():
            pltpu.sync_copy(acc.at[pl.ds(sid * RZ, RZ)],
                            out_hbm.at[0, pl.ds(sid * RZ, RZ)])

    return k(hs_pad, src3, dst3, z128)


def _dinv(deg):
    return jnp.where(deg > 0, lax.rsqrt(jnp.maximum(deg, 1e-12)), 0.0)


def _mm_scale_body(x_ref, w_ref, deg_ref, o_ref):
    o_ref[...] = (jnp.dot(x_ref[...], w_ref[...],
                          preferred_element_type=jnp.float32)
                  * _dinv(deg_ref[...]))


def _mm_scale(x, W, deg):
    return pl.pallas_call(
        _mm_scale_body,
        grid=(GRID,),
        in_specs=[
            pl.BlockSpec((BLK, D), lambda i: (i, 0)),
            pl.BlockSpec((D, D), lambda i: (0, 0)),
            pl.BlockSpec((BLK, 1), lambda i: (i, 0)),
        ],
        out_specs=pl.BlockSpec((BLK, D), lambda i: (i, 0)),
        out_shape=jax.ShapeDtypeStruct((N, D), jnp.float32),
    )(x, W, deg)


def _bn_relu_mm_scale_body(t_ref, sc_ref, sh_ref, w_ref, deg_ref, o_ref):
    r = jnp.maximum(t_ref[...] * sc_ref[...] + sh_ref[...], 0.0)
    o_ref[...] = (jnp.dot(r, w_ref[...], preferred_element_type=jnp.float32)
                  * _dinv(deg_ref[...]))


def _bn_relu_mm_scale(t, scale, shift, W, deg):
    return pl.pallas_call(
        _bn_relu_mm_scale_body,
        grid=(GRID,),
        in_specs=[
            pl.BlockSpec((BLK, D), lambda i: (i, 0)),
            pl.BlockSpec((1, D), lambda i: (0, 0)),
            pl.BlockSpec((1, D), lambda i: (0, 0)),
            pl.BlockSpec((D, D), lambda i: (0, 0)),
            pl.BlockSpec((BLK, 1), lambda i: (i, 0)),
        ],
        out_specs=pl.BlockSpec((BLK, D), lambda i: (i, 0)),
        out_shape=jax.ShapeDtypeStruct((N, D), jnp.float32),
    )(t, scale, shift, W, deg)


def _combine_stats_body(p0_ref, deg_ref, b_ref, t_ref, st_ref, acc_ref):
    i = pl.program_id(0)
    t = p0_ref[0] * _dinv(deg_ref[...]) + b_ref[...]
    t_ref[...] = t

    @pl.when(i == 0)
    def _():
        acc_ref[...] = jnp.zeros_like(acc_ref)

    acc_ref[...] += jnp.concatenate(
        [jnp.sum(t, axis=0, keepdims=True),
         jnp.sum(t * t, axis=0, keepdims=True),
         jnp.zeros((6, D), jnp.float32)], axis=0)
    st_ref[...] = acc_ref[...]


def _combine_stats(parts, deg, b):
    return pl.pallas_call(
        _combine_stats_body,
        grid=(GRID,),
        in_specs=[
            pl.BlockSpec((1, BLK, D), lambda i: (0, i, 0)),
            pl.BlockSpec((BLK, 1), lambda i: (i, 0)),
            pl.BlockSpec((1, D), lambda i: (0, 0)),
        ],
        out_specs=[
            pl.BlockSpec((BLK, D), lambda i: (i, 0)),
            pl.BlockSpec((8, D), lambda i: (0, 0)),
        ],
        out_shape=[
            jax.ShapeDtypeStruct((N, D), jnp.float32),
            jax.ShapeDtypeStruct((8, D), jnp.float32),
        ],
        scratch_shapes=[pltpu.VMEM((8, D), jnp.float32)],
    )(parts, deg, b)


def _pool_head_body(t_ref, sc_ref, sh_ref, bt_ref, wh_ref, bh_ref,
                    o_ref, ps_ref, cn_ref):
    i = pl.program_id(0)

    @pl.when(i == 0)
    def _():
        ps_ref[...] = jnp.zeros_like(ps_ref)
        cn_ref[...] = jnp.zeros_like(cn_ref)

    r = jnp.maximum(t_ref[...] * sc_ref[...] + sh_ref[...], 0.0)
    oh = (bt_ref[...] == lax.broadcasted_iota(jnp.int32, (BLK, G), 1)
          ).astype(jnp.float32)
    dn = (((0,), (0,)), ((), ()))
    ps_ref[...] += lax.dot_general(oh, r, dn,
                                   preferred_element_type=jnp.float32)
    cn_ref[...] += lax.dot_general(oh, jnp.ones_like(r), dn,
                                   preferred_element_type=jnp.float32)

    @pl.when(i == GRID - 1)
    def _():
        pooled = ps_ref[...] / jnp.maximum(cn_ref[...], 1.0)
        o_ref[...] = (jnp.dot(pooled, wh_ref[...],
                              preferred_element_type=jnp.float32)
                      + bh_ref[...])


def _pool_head(t, scale, shift, batch2, Whp, bhp):
    return pl.pallas_call(
        _pool_head_body,
        grid=(GRID,),
        in_specs=[
            pl.BlockSpec((BLK, D), lambda i: (i, 0)),
            pl.BlockSpec((1, D), lambda i: (0, 0)),
            pl.BlockSpec((1, D), lambda i: (0, 0)),
            pl.BlockSpec((BLK, 1), lambda i: (i, 0)),
            pl.BlockSpec((D, D), lambda i: (0, 0)),
            pl.BlockSpec((1, D), lambda i: (0, 0)),
        ],
        out_specs=pl.BlockSpec((G, D), lambda i: (0, 0)),
        out_shape=jax.ShapeDtypeStruct((G, D), jnp.float32),
        scratch_shapes=[pltpu.VMEM((G, D), jnp.float32),
                        pltpu.VMEM((G, D), jnp.float32)],
    )(t, scale, shift, batch2, Whp, bhp)


def kernel(x, edge_index, batch, W1, b1, g1, bt1, W2, b2, g2, bt2, Wh, bh):
    f32 = jnp.float32
    ei = jnp.concatenate(
        [edge_index, jnp.full((2, E_PAD - E), N, dtype=jnp.int32)], axis=1)
    src3 = ei[0].reshape(TCH, K)
    dst3 = ei[1].reshape(TCH, K)

    z128 = jnp.zeros((RZ, D), f32)
    padrows = jnp.zeros((N_PAD - N, D), f32)

    deg_parts = _sc_deg(dst3, z128)
    deg = (deg_parts[0, :N, 0] + deg_parts[1, :N, 0]).reshape(N, 1)

    def bn_consts(st, g, bt):
        mean = st[0] / N
        var = st[1] / N - mean * mean
        scale = g * lax.rsqrt(var + 1e-5)
        shift = bt - mean * scale
        return scale.reshape(1, D), shift.reshape(1, D)

    def do_agg(hs_pad):
        return _sc_agg(hs_pad, src3, dst3, z128)

    hs1 = _mm_scale(x, W1, deg)
    parts1 = do_agg(jnp.concatenate([hs1, padrows], axis=0))
    t1, st1 = _combine_stats(parts1, deg, b1.reshape(1, D))
    sc1, sh1 = bn_consts(st1, g1, bt1)

    hs2 = _bn_relu_mm_scale(t1, sc1, sh1, W2, deg)
    parts2 = do_agg(jnp.concatenate([hs2, padrows], axis=0))
    t2, st2 = _combine_stats(parts2, deg, b2.reshape(1, D))
    sc2, sh2 = bn_consts(st2, g2, bt2)

    Whp = jnp.concatenate([Wh, jnp.zeros((D, D - Wh.shape[1]), f32)], axis=1)
    bhp = jnp.concatenate([bh, jnp.zeros((D - bh.shape[0],), f32)]).reshape(1, D)
    out = _pool_head(t2, sc2, sh2, batch.reshape(N, 1), Whp, bhp)
    return out[:, :bh.shape[0]]


# speedup vs baseline: 1.4690x; 1.4690x over previous
import functools

import jax
import jax.numpy as jnp
from jax import lax
from jax.experimental import pallas as pl
from jax.experimental.pallas import tpu as pltpu
from jax.experimental.pallas import tpu_sc as plsc

N = 10000
E = 320000
D = 128
G = 16

NCORE = 2
NSUB = 16
NT = NCORE * NSUB

K = 128
NCH = 80
TCH = 2560
GRP = 8
NCH0 = 152
NCH1 = 8
SUPS0 = (40, 40, 40, 32)
SUPS1 = (8,)
SGMAX = 40
EPT = K * NCH
E_PAD = EPT * NT
N_PAD = 10112
RZ = N_PAD // NSUB

BLK = 1000
GRID = N // BLK


def _mesh():
    return plsc.VectorSubcoreMesh(core_axis_name="c", subcore_axis_name="s")


def _sc_deg(dst3, z128):
    @functools.partial(
        pl.kernel,
        mesh=_mesh(),
        out_type=jax.ShapeDtypeStruct((NCORE, N_PAD, D), jnp.float32),
        scratch_types=[
            pltpu.VMEM((NCH, K), jnp.int32),
            pltpu.VMEM((K, D), jnp.float32),
            pltpu.VMEM_SHARED((N_PAD, D), jnp.float32),
        ],
    )
    def k(dst_hbm, z_hbm, out_hbm, dst_v, ones_v, acc):
        cid = lax.axis_index("c")
        sid = lax.axis_index("s")
        wid = sid * NCORE + cid
        pltpu.sync_copy(z_hbm, acc.at[pl.ds(sid * RZ, RZ)])

        def fill(r, carry):
            for g in range(D // 16):
                ones_v[r, pl.ds(g * 16, 16)] = jnp.full((16,), 1.0,
                                                        jnp.float32)
            return carry

        lax.fori_loop(0, K, fill, 0)
        pltpu.sync_copy(dst_hbm.at[pl.ds(wid * NCH, NCH)], dst_v)
        plsc.subcore_barrier()

        def body(c, carry):
            pltpu.sync_copy(ones_v, acc.at[dst_v.at[c]], add=True)
            return carry

        lax.fori_loop(0, NCH, body, 0)
        plsc.subcore_barrier()
        pltpu.sync_copy(acc.at[pl.ds(sid * RZ, RZ)],
                        out_hbm.at[cid, pl.ds(sid * RZ, RZ)])

    return k(dst3, z128)


def _sc_agg(hs_pad, src3, dst3, z128):
    @functools.partial(
        pl.kernel,
        mesh=_mesh(),
        out_type=jax.ShapeDtypeStruct((NCORE, N_PAD, D), jnp.float32),
        scratch_types=[
            pltpu.VMEM((SGMAX, K), jnp.int32),
            pltpu.VMEM((SGMAX, K), jnp.int32),
            pltpu.VMEM((K, D), jnp.float32),
            pltpu.VMEM((K, D), jnp.float32),
            pltpu.VMEM_SHARED((N_PAD, D), jnp.float32),
            pltpu.SemaphoreType.DMA,
            pltpu.SemaphoreType.DMA,
        ],
    )
    def k(hs_hbm, src_hbm, dst_hbm, z_hbm, out_hbm,
          ssup, dsup, r0, r1, acc, s0, s1):
        cid = lax.axis_index("c")
        sid = lax.axis_index("s")
        pltpu.sync_copy(z_hbm, acc.at[pl.ds(sid * RZ, RZ)])
        plsc.subcore_barrier()

        rows = (r0, r1)
        sems = (s0, s1)

        def run_part(base, supers):
            off = 0
            for sg in supers:
                sb = base + off
                pltpu.sync_copy(src_hbm.at[pl.ds(sb, sg)],
                                ssup.at[pl.ds(0, sg)])
                pltpu.sync_copy(dst_hbm.at[pl.ds(sb, sg)],
                                dsup.at[pl.ds(0, sg)])
                pltpu.async_copy(hs_hbm.at[ssup.at[0]], r0, s0)
                pltpu.async_copy(hs_hbm.at[ssup.at[1]], r1, s1)

                def pair(j, carry):
                    for b in range(2):
                        c = 2 * j + b
                        pltpu.make_async_copy(hs_hbm.at[ssup.at[c]],
                                              rows[b], sems[b]).wait()
                        pltpu.sync_copy(rows[b], acc.at[dsup.at[c]],
                                        add=True)

                        @pl.when(c + 2 < sg)
                        def _():
                            pltpu.async_copy(hs_hbm.at[ssup.at[c + 2]],
                                             rows[b], sems[b])
                    return carry

                lax.fori_loop(0, sg // 2, pair, 0)
                off += sg

        @pl.when(cid == 0)
        def _():
            run_part(sid * NCH0, SUPS0)

        @pl.when(cid == 1)
        def _():
            run_part(NSUB * NCH0 + sid * NCH1, SUPS1)

        plsc.subcore_barrier()
        pltpu.sync_copy(acc.at[pl.ds(sid * RZ, RZ)],
                        out_hbm.at[cid, pl.ds(sid * RZ, RZ)])

    return k(hs_pad, src3, dst3, z128)


def _dinv(deg):
    return jnp.where(deg > 0, lax.rsqrt(jnp.maximum(deg, 1e-12)), 0.0)


def _mm_scale_body(x_ref, w_ref, deg_ref, o_ref):
    o_ref[...] = (jnp.dot(x_ref[...], w_ref[...],
                          preferred_element_type=jnp.float32)
                  * _dinv(deg_ref[...]))


def _mm_scale(x, W, deg):
    return pl.pallas_call(
        _mm_scale_body,
        grid=(GRID,),
        in_specs=[
            pl.BlockSpec((BLK, D), lambda i: (i, 0)),
            pl.BlockSpec((D, D), lambda i: (0, 0)),
            pl.BlockSpec((BLK, 1), lambda i: (i, 0)),
        ],
        out_specs=pl.BlockSpec((BLK, D), lambda i: (i, 0)),
        out_shape=jax.ShapeDtypeStruct((N, D), jnp.float32),
    )(x, W, deg)


def _bn_relu_mm_scale_body(t_ref, sc_ref, sh_ref, w_ref, deg_ref, o_ref):
    r = jnp.maximum(t_ref[...] * sc_ref[...] + sh_ref[...], 0.0)
    o_ref[...] = (jnp.dot(r, w_ref[...], preferred_element_type=jnp.float32)
                  * _dinv(deg_ref[...]))


def _bn_relu_mm_scale(t, scale, shift, W, deg):
    return pl.pallas_call(
        _bn_relu_mm_scale_body,
        grid=(GRID,),
        in_specs=[
            pl.BlockSpec((BLK, D), lambda i: (i, 0)),
            pl.BlockSpec((1, D), lambda i: (0, 0)),
            pl.BlockSpec((1, D), lambda i: (0, 0)),
            pl.BlockSpec((D, D), lambda i: (0, 0)),
            pl.BlockSpec((BLK, 1), lambda i: (i, 0)),
        ],
        out_specs=pl.BlockSpec((BLK, D), lambda i: (i, 0)),
        out_shape=jax.ShapeDtypeStruct((N, D), jnp.float32),
    )(t, scale, shift, W, deg)


def _combine_stats_body(p0_ref, p1_ref, deg_ref, b_ref, t_ref, st_ref,
                        acc_ref):
    i = pl.program_id(0)
    t = (p0_ref[0] + p1_ref[0]) * _dinv(deg_ref[...]) + b_ref[...]
    t_ref[...] = t

    @pl.when(i == 0)
    def _():
        acc_ref[...] = jnp.zeros_like(acc_ref)

    acc_ref[...] += jnp.concatenate(
        [jnp.sum(t, axis=0, keepdims=True),
         jnp.sum(t * t, axis=0, keepdims=True),
         jnp.zeros((6, D), jnp.float32)], axis=0)
    st_ref[...] = acc_ref[...]


def _combine_stats(parts, deg, b):
    return pl.pallas_call(
        _combine_stats_body,
        grid=(GRID,),
        in_specs=[
            pl.BlockSpec((1, BLK, D), lambda i: (0, i, 0)),
            pl.BlockSpec((1, BLK, D), lambda i: (1, i, 0)),
            pl.BlockSpec((BLK, 1), lambda i: (i, 0)),
            pl.BlockSpec((1, D), lambda i: (0, 0)),
        ],
        out_specs=[
            pl.BlockSpec((BLK, D), lambda i: (i, 0)),
            pl.BlockSpec((8, D), lambda i: (0, 0)),
        ],
        out_shape=[
            jax.ShapeDtypeStruct((N, D), jnp.float32),
            jax.ShapeDtypeStruct((8, D), jnp.float32),
        ],
        scratch_shapes=[pltpu.VMEM((8, D), jnp.float32)],
    )(parts, parts, deg, b)


def _pool_head_body(t_ref, sc_ref, sh_ref, bt_ref, wh_ref, bh_ref,
                    o_ref, ps_ref, cn_ref):
    i = pl.program_id(0)

    @pl.when(i == 0)
    def _():
        ps_ref[...] = jnp.zeros_like(ps_ref)
        cn_ref[...] = jnp.zeros_like(cn_ref)

    r = jnp.maximum(t_ref[...] * sc_ref[...] + sh_ref[...], 0.0)
    oh = (bt_ref[...] == lax.broadcasted_iota(jnp.int32, (BLK, G), 1)
          ).astype(jnp.float32)
    dn = (((0,), (0,)), ((), ()))
    ps_ref[...] += lax.dot_general(oh, r, dn,
                                   preferred_element_type=jnp.float32)
    cn_ref[...] += lax.dot_general(oh, jnp.ones_like(r), dn,
                                   preferred_element_type=jnp.float32)

    @pl.when(i == GRID - 1)
    def _():
        pooled = ps_ref[...] / jnp.maximum(cn_ref[...], 1.0)
        o_ref[...] = (jnp.dot(pooled, wh_ref[...],
                              preferred_element_type=jnp.float32)
                      + bh_ref[...])


def _pool_head(t, scale, shift, batch2, Whp, bhp):
    return pl.pallas_call(
        _pool_head_body,
        grid=(GRID,),
        in_specs=[
            pl.BlockSpec((BLK, D), lambda i: (i, 0)),
            pl.BlockSpec((1, D), lambda i: (0, 0)),
            pl.BlockSpec((1, D), lambda i: (0, 0)),
            pl.BlockSpec((BLK, 1), lambda i: (i, 0)),
            pl.BlockSpec((D, D), lambda i: (0, 0)),
            pl.BlockSpec((1, D), lambda i: (0, 0)),
        ],
        out_specs=pl.BlockSpec((G, D), lambda i: (0, 0)),
        out_shape=jax.ShapeDtypeStruct((G, D), jnp.float32),
        scratch_shapes=[pltpu.VMEM((G, D), jnp.float32),
                        pltpu.VMEM((G, D), jnp.float32)],
    )(t, scale, shift, batch2, Whp, bhp)


def kernel(x, edge_index, batch, W1, b1, g1, bt1, W2, b2, g2, bt2, Wh, bh):
    f32 = jnp.float32
    ei = jnp.concatenate(
        [edge_index, jnp.full((2, E_PAD - E), N, dtype=jnp.int32)], axis=1)
    src3 = ei[0].reshape(TCH, K)
    dst3 = ei[1].reshape(TCH, K)

    z128 = jnp.zeros((RZ, D), f32)
    padrows = jnp.zeros((N_PAD - N, D), f32)

    deg_parts = _sc_deg(dst3, z128)
    deg = (deg_parts[0, :N, 0] + deg_parts[1, :N, 0]).reshape(N, 1)

    def bn_consts(st, g, bt):
        mean = st[0] / N
        var = st[1] / N - mean * mean
        scale = g * lax.rsqrt(var + 1e-5)
        shift = bt - mean * scale
        return scale.reshape(1, D), shift.reshape(1, D)

    def do_agg(hs_pad):
        return _sc_agg(hs_pad, src3, dst3, z128)

    hs1 = _mm_scale(x, W1, deg)
    parts1 = do_agg(jnp.concatenate([hs1, padrows], axis=0))
    t1, st1 = _combine_stats(parts1, deg, b1.reshape(1, D))
    sc1, sh1 = bn_consts(st1, g1, bt1)

    hs2 = _bn_relu_mm_scale(t1, sc1, sh1, W2, deg)
    parts2 = do_agg(jnp.concatenate([hs2, padrows], axis=0))
    t2, st2 = _combine_stats(parts2, deg, b2.reshape(1, D))
    sc2, sh2 = bn_consts(st2, g2, bt2)

    Whp = jnp.concatenate([Wh, jnp.zeros((D, D - Wh.shape[1]), f32)], axis=1)
    bhp = jnp.concatenate([bh, jnp.zeros((D - bh.shape[0],), f32)]).reshape(1, D)
    out = _pool_head(t2, sc2, sh2, batch.reshape(N, 1), Whp, bhp)
    return out[:, :bh.shape[0]]


# speedup vs baseline: 1.4691x; 1.0000x over previous
import functools

import jax
import jax.numpy as jnp
from jax import lax
from jax.experimental import pallas as pl
from jax.experimental.pallas import tpu as pltpu
from jax.experimental.pallas import tpu_sc as plsc

N = 10000
E = 320000
D = 128
G = 16

NCORE = 2
NSUB = 16
NT = NCORE * NSUB

K = 128
NCH = 80
TCH = 2560
NCH0 = 152
NCH1 = 8
SUPS0 = (40, 40, 40, 32)
SUPS1 = (8,)
SGMAX = 40
EPT = K * NCH
E_PAD = EPT * NT
N_PAD = 10112
RZ = N_PAD // NSUB

BLK = 1000
GRID = N // BLK


def _mesh():
    return plsc.VectorSubcoreMesh(core_axis_name="c", subcore_axis_name="s")


def _sc_deg(dst3, z128):
    @functools.partial(
        pl.kernel,
        mesh=_mesh(),
        out_type=jax.ShapeDtypeStruct((NCORE, N_PAD, D), jnp.float32),
        scratch_types=[
            pltpu.VMEM((NCH, K), jnp.int32),
            pltpu.VMEM((K, D), jnp.float32),
            pltpu.VMEM_SHARED((N_PAD, D), jnp.float32),
        ],
    )
    def k(dst_hbm, z_hbm, out_hbm, dst_v, ones_v, acc):
        cid = lax.axis_index("c")
        sid = lax.axis_index("s")
        wid = sid * NCORE + cid
        pltpu.sync_copy(z_hbm, acc.at[pl.ds(sid * RZ, RZ)])

        def fill(r, carry):
            for g in range(D // 16):
                ones_v[r, pl.ds(g * 16, 16)] = jnp.full((16,), 1.0,
                                                        jnp.float32)
            return carry

        lax.fori_loop(0, K, fill, 0)
        pltpu.sync_copy(dst_hbm.at[pl.ds(wid * NCH, NCH)], dst_v)
        plsc.subcore_barrier()

        def body(c, carry):
            pltpu.sync_copy(ones_v, acc.at[dst_v.at[c]], add=True)
            return carry

        lax.fori_loop(0, NCH, body, 0)
        plsc.subcore_barrier()
        pltpu.sync_copy(acc.at[pl.ds(sid * RZ, RZ)],
                        out_hbm.at[cid, pl.ds(sid * RZ, RZ)])

    return k(dst3, z128)


def _sc_agg(hs_pad, src3, dst3, z128):
    @functools.partial(
        pl.kernel,
        mesh=_mesh(),
        out_type=jax.ShapeDtypeStruct((NCORE, N_PAD, D), jnp.float32),
        scratch_types=[
            pltpu.VMEM((SGMAX, K), jnp.int32),
            pltpu.VMEM((SGMAX, K), jnp.int32),
            pltpu.VMEM((K, D), jnp.float32),
            pltpu.VMEM((K, D), jnp.float32),
            pltpu.VMEM_SHARED((N_PAD, D), jnp.float32),
            pltpu.SemaphoreType.DMA,
            pltpu.SemaphoreType.DMA,
        ],
    )
    def k(hs_hbm, src_hbm, dst_hbm, z_hbm, out_hbm,
          ssup, dsup, r0, r1, acc, s0, s1):
        cid = lax.axis_index("c")
        sid = lax.axis_index("s")
        pltpu.sync_copy(z_hbm, acc.at[pl.ds(sid * RZ, RZ)])
        plsc.subcore_barrier()

        rows = (r0, r1)
        sems = (s0, s1)

        def run_part(base, supers):
            off = 0
            for sg in supers:
                sb = base + off
                pltpu.sync_copy(src_hbm.at[pl.ds(sb, sg)],
                                ssup.at[pl.ds(0, sg)])
                pltpu.sync_copy(dst_hbm.at[pl.ds(sb, sg)],
                                dsup.at[pl.ds(0, sg)])
                pltpu.async_copy(hs_hbm.at[ssup.at[0]], r0, s0)
                pltpu.async_copy(hs_hbm.at[ssup.at[1]], r1, s1)

                def pair(j, carry):
                    for b in range(2):
                        c = 2 * j + b
                        pltpu.make_async_copy(hs_hbm.at[ssup.at[c]],
                                              rows[b], sems[b]).wait()
                        pltpu.sync_copy(rows[b], acc.at[dsup.at[c]],
                                        add=True)

                        @pl.when(c + 2 < sg)
                        def _():
                            pltpu.async_copy(hs_hbm.at[ssup.at[c + 2]],
                                             rows[b], sems[b])
                    return carry

                lax.fori_loop(0, sg // 2, pair, 0)
                off += sg

        @pl.when(cid == 0)
        def _():
            run_part(sid * NCH0, SUPS0)

        @pl.when(cid == 1)
        def _():
            run_part(NSUB * NCH0 + sid * NCH1, SUPS1)

        plsc.subcore_barrier()
        pltpu.sync_copy(acc.at[pl.ds(sid * RZ, RZ)],
                        out_hbm.at[cid, pl.ds(sid * RZ, RZ)])

    return k(hs_pad, src3, dst3, z128)


def _dinv(deg):
    return jnp.where(deg > 0, lax.rsqrt(jnp.maximum(deg, 1e-12)), 0.0)


def _mm_scale_body(x_ref, w_ref, deg_ref, o_ref):
    o_ref[...] = (jnp.dot(x_ref[...], w_ref[...],
                          preferred_element_type=jnp.float32)
                  * _dinv(deg_ref[...]))


def _mm_scale(x, W, deg):
    return pl.pallas_call(
        _mm_scale_body,
        grid=(GRID,),
        in_specs=[
            pl.BlockSpec((BLK, D), lambda i: (i, 0)),
            pl.BlockSpec((D, D), lambda i: (0, 0)),
            pl.BlockSpec((BLK, 1), lambda i: (i, 0)),
        ],
        out_specs=pl.BlockSpec((BLK, D), lambda i: (i, 0)),
        out_shape=jax.ShapeDtypeStruct((N, D), jnp.float32),
    )(x, W, deg)


def _bn_relu_mm_scale_body(t_ref, sc_ref, sh_ref, w_ref, deg_ref, o_ref):
    r = jnp.maximum(t_ref[...] * sc_ref[...] + sh_ref[...], 0.0)
    o_ref[...] = (jnp.dot(r, w_ref[...], preferred_element_type=jnp.float32)
                  * _dinv(deg_ref[...]))


def _bn_relu_mm_scale(t, scale, shift, W, deg):
    return pl.pallas_call(
        _bn_relu_mm_scale_body,
        grid=(GRID,),
        in_specs=[
            pl.BlockSpec((BLK, D), lambda i: (i, 0)),
            pl.BlockSpec((1, D), lambda i: (0, 0)),
            pl.BlockSpec((1, D), lambda i: (0, 0)),
            pl.BlockSpec((D, D), lambda i: (0, 0)),
            pl.BlockSpec((BLK, 1), lambda i: (i, 0)),
        ],
        out_specs=pl.BlockSpec((BLK, D), lambda i: (i, 0)),
        out_shape=jax.ShapeDtypeStruct((N, D), jnp.float32),
    )(t, scale, shift, W, deg)


def _combine_stats_body(p0_ref, p1_ref, deg_ref, b_ref, t_ref, st_ref,
                        acc_ref):
    i = pl.program_id(0)
    t = (p0_ref[0] + p1_ref[0]) * _dinv(deg_ref[...]) + b_ref[...]
    t_ref[...] = t

    @pl.when(i == 0)
    def _():
        acc_ref[...] = jnp.zeros_like(acc_ref)

    acc_ref[...] += jnp.concatenate(
        [jnp.sum(t, axis=0, keepdims=True),
         jnp.sum(t * t, axis=0, keepdims=True),
         jnp.zeros((6, D), jnp.float32)], axis=0)
    st_ref[...] = acc_ref[...]


def _combine_stats(parts, deg, b):
    return pl.pallas_call(
        _combine_stats_body,
        grid=(GRID,),
        in_specs=[
            pl.BlockSpec((1, BLK, D), lambda i: (0, i, 0)),
            pl.BlockSpec((1, BLK, D), lambda i: (1, i, 0)),
            pl.BlockSpec((BLK, 1), lambda i: (i, 0)),
            pl.BlockSpec((1, D), lambda i: (0, 0)),
        ],
        out_specs=[
            pl.BlockSpec((BLK, D), lambda i: (i, 0)),
            pl.BlockSpec((8, D), lambda i: (0, 0)),
        ],
        out_shape=[
            jax.ShapeDtypeStruct((N, D), jnp.float32),
            jax.ShapeDtypeStruct((8, D), jnp.float32),
        ],
        scratch_shapes=[pltpu.VMEM((8, D), jnp.float32)],
    )(parts, parts, deg, b)


def _pool_head_body(t_ref, sc_ref, sh_ref, bt_ref, wh_ref, bh_ref,
                    o_ref, ps_ref, cn_ref):
    i = pl.program_id(0)

    @pl.when(i == 0)
    def _():
        ps_ref[...] = jnp.zeros_like(ps_ref)
        cn_ref[...] = jnp.zeros_like(cn_ref)

    r = jnp.maximum(t_ref[...] * sc_ref[...] + sh_ref[...], 0.0)
    oh = (bt_ref[...] == lax.broadcasted_iota(jnp.int32, (BLK, G), 1)
          ).astype(jnp.float32)
    dn = (((0,), (0,)), ((), ()))
    ps_ref[...] += lax.dot_general(oh, r, dn,
                                   preferred_element_type=jnp.float32)
    cn_ref[...] += lax.dot_general(oh, jnp.ones_like(r), dn,
                                   preferred_element_type=jnp.float32)

    @pl.when(i == GRID - 1)
    def _():
        pooled = ps_ref[...] / jnp.maximum(cn_ref[...], 1.0)
        o_ref[...] = (jnp.dot(pooled, wh_ref[...],
                              preferred_element_type=jnp.float32)
                      + bh_ref[...])


def _pool_head(t, scale, shift, batch2, Whp, bhp):
    return pl.pallas_call(
        _pool_head_body,
        grid=(GRID,),
        in_specs=[
            pl.BlockSpec((BLK, D), lambda i: (i, 0)),
            pl.BlockSpec((1, D), lambda i: (0, 0)),
            pl.BlockSpec((1, D), lambda i: (0, 0)),
            pl.BlockSpec((BLK, 1), lambda i: (i, 0)),
            pl.BlockSpec((D, D), lambda i: (0, 0)),
            pl.BlockSpec((1, D), lambda i: (0, 0)),
        ],
        out_specs=pl.BlockSpec((G, D), lambda i: (0, 0)),
        out_shape=jax.ShapeDtypeStruct((G, D), jnp.float32),
        scratch_shapes=[pltpu.VMEM((G, D), jnp.float32),
                        pltpu.VMEM((G, D), jnp.float32)],
    )(t, scale, shift, batch2, Whp, bhp)


def kernel(x, edge_index, batch, W1, b1, g1, bt1, W2, b2, g2, bt2, Wh, bh):
    f32 = jnp.float32
    ei = jnp.concatenate(
        [edge_index, jnp.full((2, E_PAD - E), N, dtype=jnp.int32)], axis=1)
    src3 = ei[0].reshape(TCH, K)
    dst3 = ei[1].reshape(TCH, K)

    z128 = jnp.zeros((RZ, D), f32)
    padrows = jnp.zeros((N_PAD - N, D), f32)

    deg_parts = _sc_deg(dst3, z128)
    deg = (deg_parts[0, :N, 0] + deg_parts[1, :N, 0]).reshape(N, 1)

    def bn_consts(st, g, bt):
        mean = st[0] / N
        var = st[1] / N - mean * mean
        scale = g * lax.rsqrt(var + 1e-5)
        shift = bt - mean * scale
        return scale.reshape(1, D), shift.reshape(1, D)

    def do_agg(hs_pad):
        return _sc_agg(hs_pad, src3, dst3, z128)

    hs1 = _mm_scale(x, W1, deg)
    parts1 = do_agg(jnp.concatenate([hs1, padrows], axis=0))
    t1, st1 = _combine_stats(parts1, deg, b1.reshape(1, D))
    sc1, sh1 = bn_consts(st1, g1, bt1)

    hs2 = _bn_relu_mm_scale(t1, sc1, sh1, W2, deg)
    parts2 = do_agg(jnp.concatenate([hs2, padrows], axis=0))
    t2, st2 = _combine_stats(parts2, deg, b2.reshape(1, D))
    sc2, sh2 = bn_consts(st2, g2, bt2)

    Whp = jnp.concatenate([Wh, jnp.zeros((D, D - Wh.shape[1]), f32)], axis=1)
    bhp = jnp.concatenate([bh, jnp.zeros((D - bh.shape[0],), f32)]).reshape(1, D)
    out = _pool_head(t2, sc2, sh2, batch.reshape(N, 1), Whp, bhp)
    return out[:, :bh.shape[0]]
